# baseline (speedup 1.0000x reference)
import functools

import jax
import jax.numpy as jnp
from jax.experimental import pallas as pl
from jax.experimental.pallas import tpu as pltpu


def _write_kernel(emb_hbm, out_hbm, sbuf, sem, *, rows, scale):
    sbuf[...] = jnp.full_like(sbuf, scale)
    pltpu.make_async_copy(sbuf, out_hbm.at[pl.ds(0, rows)], sem).start()
    pltpu.make_async_copy(sbuf, out_hbm.at[pl.ds(0, rows)], sem).wait()


def kernel(x, emb_weight, pos):
    del x, pos
    max_seq_len, dim = emb_weight.shape
    dtype = emb_weight.dtype
    rows = 256
    emb3 = emb_weight.reshape(max_seq_len, 1, dim)
    out = pl.pallas_call(
        functools.partial(_write_kernel, rows=rows, scale=0.5),
        grid=(1,),
        in_specs=[pl.BlockSpec(memory_space=pl.ANY)],
        out_specs=pl.BlockSpec(memory_space=pl.ANY),
        scratch_shapes=[pltpu.VMEM((rows, 1, dim), dtype),
                        pltpu.SemaphoreType.DMA],
        out_shape=jax.ShapeDtypeStruct((rows, 1, dim), dtype),
        compiler_params=pltpu.CompilerParams(
            dimension_semantics=("arbitrary",),
            vmem_limit_bytes=int(32 << 20)),
    )(emb_weight.reshape(max_seq_len * 8, dim // 8))
    return out.reshape(rows, dim)


# speedup vs baseline: 1.1789x; 1.1789x over previous
import functools

import jax
import jax.numpy as jnp
from jax.experimental import pallas as pl
from jax.experimental.pallas import tpu as pltpu


def _gather_kernel(pos_ref, emb_hbm, out_ref, tbl, sem, *, rows, scale):
    j = pl.program_id(0)
    n, _, d = tbl.shape

    @pl.when(j == 0)
    def _():
        cp = pltpu.make_async_copy(emb_hbm, tbl.reshape(n, d), sem)
        cp.start()
        cp.wait()

    base = j * rows
    for mi in range(rows):
        p = pos_ref[base + mi]
        out_ref[mi, 0, :] = tbl[p, 0, :] * scale


def _gather(emb_weight, pos, rows=256):
    max_seq_len, dim = emb_weight.shape
    dtype = emb_weight.dtype
    scale = dim ** (-0.5)
    pos = pos.astype(jnp.int32)
    out_len = pos.shape[0]

    padded = ((out_len + rows - 1) // rows) * rows
    if padded != out_len:
        pos = jnp.concatenate(
            [pos, jnp.zeros((padded - out_len,), jnp.int32)])
    n_blocks = padded // rows

    table_bytes = max_seq_len * dim * jnp.dtype(dtype).itemsize
    block_bytes = rows * dim * jnp.dtype(dtype).itemsize
    vmem_limit = int(min(60 << 20, table_bytes + 4 * block_bytes + (4 << 20)))

    out = pl.pallas_call(
        functools.partial(_gather_kernel, rows=rows, scale=scale),
        grid_spec=pltpu.PrefetchScalarGridSpec(
            num_scalar_prefetch=1,
            grid=(n_blocks,),
            in_specs=[pl.BlockSpec(memory_space=pl.ANY)],
            out_specs=pl.BlockSpec(
                (rows, 1, dim), lambda j, pos_ref: (j, 0, 0)),
            scratch_shapes=[pltpu.VMEM((max_seq_len, 1, dim), dtype),
                            pltpu.SemaphoreType.DMA],
        ),
        out_shape=jax.ShapeDtypeStruct((padded, 1, dim), dtype),
        compiler_params=pltpu.CompilerParams(
            dimension_semantics=("arbitrary",),
            vmem_limit_bytes=vmem_limit),
    )(pos, emb_weight)
    return out[:out_len].reshape(out_len, dim)


def kernel(x, emb_weight, pos):
    del x
    return _gather(emb_weight, pos)
